# baseline (speedup 1.0000x reference)
import jax
import jax.numpy as jnp
from jax import lax
from jax.experimental import pallas as pl
from jax.experimental.pallas import tpu as pltpu

ROWS, COLS = 128, 32768
CC = 2048
NCHUNK = COLS // CC


def _tc_body(x_hbm, o_ref, buf, sem0, sem1):
    sems = (sem0, sem1)

    def start(c):
        return pltpu.make_async_copy(
            x_hbm.at[:, pl.ds(c * CC, CC)], buf.at[c % 2], sems[c % 2]
        )

    start(0).start()
    runmax = jnp.full((ROWS, 1), float("-inf"), jnp.float32)
    runidx = jnp.zeros((ROWS,), jnp.int32)
    iota = lax.broadcasted_iota(jnp.int32, (ROWS, CC), 1)

    for c in range(NCHUNK):
        if c + 1 < NCHUNK:
            start(c + 1).start()
        start(c).wait()
        chunk = buf[c % 2]
        cmax = jnp.max(chunk, axis=1, keepdims=True)
        cidx = jnp.min(jnp.where(chunk == cmax, iota, COLS), axis=1) + c * CC
        better = cmax > runmax
        runidx = jnp.where(better[:, 0], cidx, runidx)
        runmax = jnp.maximum(runmax, cmax)

    o_ref[...] = runidx


def _argmax_tc(x):
    return pl.pallas_call(
        _tc_body,
        in_specs=[pl.BlockSpec(memory_space=pl.ANY)],
        out_specs=pl.BlockSpec(memory_space=pltpu.MemorySpace.VMEM),
        out_shape=jax.ShapeDtypeStruct((ROWS,), jnp.int32),
        scratch_shapes=[
            pltpu.VMEM((2, ROWS, CC), jnp.float32),
            pltpu.SemaphoreType.DMA,
            pltpu.SemaphoreType.DMA,
        ],
    )(x)


def kernel(x):
    return _argmax_tc(x)


# speedup vs baseline: 1.2784x; 1.2784x over previous
import jax
import jax.numpy as jnp
from jax import lax
from jax.experimental import pallas as pl
from jax.experimental.pallas import tpu as pltpu

ROWS, COLS = 128, 32768
CR = 16
NCHUNK = ROWS // CR


def _tc_body(x_hbm, o_ref, buf, sem0, sem1):
    sems = (sem0, sem1)

    def start(c):
        return pltpu.make_async_copy(
            x_hbm.at[pl.ds(c * CR, CR), :], buf.at[c % 2], sems[c % 2]
        )

    start(0).start()
    iota = lax.broadcasted_iota(jnp.int32, (CR, COLS), 1)

    for c in range(NCHUNK):
        if c + 1 < NCHUNK:
            start(c + 1).start()
        start(c).wait()
        xb = buf[c % 2]
        m = jnp.max(xb, axis=1, keepdims=True)
        idx = jnp.where(xb == m, iota, COLS)
        o_ref[pl.ds(c * CR, CR)] = jnp.min(idx, axis=1)


def _argmax_tc(x):
    return pl.pallas_call(
        _tc_body,
        in_specs=[pl.BlockSpec(memory_space=pl.ANY)],
        out_specs=pl.BlockSpec(memory_space=pltpu.MemorySpace.VMEM),
        out_shape=jax.ShapeDtypeStruct((ROWS,), jnp.int32),
        scratch_shapes=[
            pltpu.VMEM((2, CR, COLS), jnp.float32),
            pltpu.SemaphoreType.DMA,
            pltpu.SemaphoreType.DMA,
        ],
    )(x)


def kernel(x):
    return _argmax_tc(x)


# speedup vs baseline: 1.4949x; 1.1694x over previous
import jax
import jax.numpy as jnp
from jax import lax
from jax.experimental import pallas as pl
from jax.experimental.pallas import tpu as pltpu

ROWS, COLS = 128, 32768
CR = 16
NCHUNK = ROWS // CR
NBUF = 4


def _tc_body(x_hbm, o_ref, buf, sem0, sem1, sem2, sem3):
    sems = (sem0, sem1, sem2, sem3)

    def start(c):
        return pltpu.make_async_copy(
            x_hbm.at[pl.ds(c * CR, CR), :], buf.at[c % NBUF], sems[c % NBUF]
        )

    for p in range(NBUF - 1):
        start(p).start()
    iota = lax.broadcasted_iota(jnp.int32, (CR, COLS), 1)

    for c in range(NCHUNK):
        if c + NBUF - 1 < NCHUNK:
            start(c + NBUF - 1).start()
        start(c).wait()
        xb = buf[c % NBUF]
        m = jnp.max(xb, axis=1, keepdims=True)
        idx = jnp.where(xb == m, iota, COLS)
        o_ref[pl.ds(c * CR, CR)] = jnp.min(idx, axis=1)


def _argmax_tc(x):
    return pl.pallas_call(
        _tc_body,
        in_specs=[pl.BlockSpec(memory_space=pl.ANY)],
        out_specs=pl.BlockSpec(memory_space=pltpu.MemorySpace.VMEM),
        out_shape=jax.ShapeDtypeStruct((ROWS,), jnp.int32),
        scratch_shapes=[
            pltpu.VMEM((NBUF, CR, COLS), jnp.float32),
            pltpu.SemaphoreType.DMA,
            pltpu.SemaphoreType.DMA,
            pltpu.SemaphoreType.DMA,
            pltpu.SemaphoreType.DMA,
        ],
    )(x)


def kernel(x):
    return _argmax_tc(x)
